# baseline (device time: 42720 ns/iter reference)
import jax
import jax.numpy as jnp
from jax import lax
from jax.experimental import pallas as pl
from jax.experimental.pallas import tpu as pltpu

N_DEV = 32


def kernel(Q, K, V):
    b, q_len, h, d = Q.shape
    kv_len = K.shape[1]
    kh = kv_len * h
    bh = b * h
    scale = d ** -0.5
    pack = 128

    Q3 = Q.reshape(b, h, d)
    K3 = K.reshape(b, kh, d)
    V3 = V.reshape(b, kh, d)

    def body(q_ref, k_ref, v_ref, o_ref, send_ref, allrecv_ref, send_sems, recv_sems):
        my = lax.axis_index("i")

        barrier_sem = pltpu.get_barrier_semaphore()
        for off in range(1, N_DEV):
            pl.semaphore_signal(
                barrier_sem,
                inc=1,
                device_id=((my + off) % N_DEV,),
                device_id_type=pl.DeviceIdType.MESH,
            )

        f32 = jnp.float32
        neg = jnp.where(
            lax.broadcasted_iota(jnp.int32, (h, kh), 1) % h
            == lax.broadcasted_iota(jnp.int32, (h, kh), 0),
            jnp.zeros((h, kh), f32),
            jnp.full((h, kh), -1e30, f32),
        )

        qall = q_ref[...] * scale
        o_rows, m_rows, l_rows = [], [], []
        for bi in range(b):
            St = lax.dot_general(
                qall[bi], k_ref[bi], (((1,), (1,)), ((), ())),
                preferred_element_type=f32,
            ) + neg
            mb = jnp.max(St, axis=1, keepdims=True)
            pb = jnp.exp(St - mb)
            lb = jnp.sum(pb, axis=1, keepdims=True)
            Ob = lax.dot_general(
                pb, v_ref[bi], (((1,), (0,)), ((), ())),
                preferred_element_type=f32,
            )
            o_rows.append(Ob)
            m_rows.append(mb)
            l_rows.append(lb)

        o = jnp.concatenate(o_rows, axis=0)
        m = jnp.concatenate(m_rows, axis=0)
        l = jnp.concatenate(l_rows, axis=0)

        send_ref[:, 0:d] = o.astype(jnp.bfloat16)
        send_ref[:, d:d + 1] = m.astype(jnp.bfloat16)
        send_ref[:, d + 1:d + 2] = l.astype(jnp.bfloat16)

        pl.semaphore_wait(barrier_sem, N_DEV - 1)

        sends = []
        for off in range(1, N_DEV):
            rdma = pltpu.make_async_remote_copy(
                src_ref=send_ref,
                dst_ref=allrecv_ref.at[my],
                send_sem=send_sems.at[off],
                recv_sem=recv_sems.at[my],
                device_id=((my + off) % N_DEV,),
                device_id_type=pl.DeviceIdType.MESH,
            )
            rdma.start()
            sends.append(rdma)

        allrecv_ref[my] = send_ref[...]

        for off in range(1, N_DEV):
            src = (my + off) % N_DEV
            recv = pltpu.make_async_remote_copy(
                src_ref=send_ref,
                dst_ref=allrecv_ref.at[src],
                send_sem=send_sems.at[off],
                recv_sem=recv_sems.at[src],
                device_id=(src,),
                device_id_type=pl.DeviceIdType.MESH,
            )
            recv.wait_recv()

        data = allrecv_ref[...]
        om = data[:, :, 0:d].astype(f32)
        mm = data[:, :, d:d + 1].astype(f32)
        lm = data[:, :, d + 1:d + 2].astype(f32)

        mg = jnp.max(mm, axis=0)
        a = jnp.exp(mm - mg[None, :, :])
        lg = jnp.sum(lm * a, axis=0)
        og = jnp.sum(om * a, axis=0)
        o_ref[...] = og / lg

        for rdma in sends:
            rdma.wait_send()

    out2d = pl.pallas_call(
        body,
        out_shape=jax.ShapeDtypeStruct((bh, d), jnp.float32),
        in_specs=[
            pl.BlockSpec(memory_space=pltpu.VMEM),
            pl.BlockSpec(memory_space=pltpu.VMEM),
            pl.BlockSpec(memory_space=pltpu.VMEM),
        ],
        out_specs=pl.BlockSpec(memory_space=pltpu.VMEM),
        scratch_shapes=[
            pltpu.VMEM((bh, pack), jnp.bfloat16),
            pltpu.VMEM((N_DEV, bh, pack), jnp.bfloat16),
            pltpu.SemaphoreType.DMA((N_DEV,)),
            pltpu.SemaphoreType.DMA((N_DEV,)),
        ],
        compiler_params=pltpu.CompilerParams(collective_id=0),
    )(Q3, K3, V3)
    return out2d.reshape(b, q_len, h, d)


# device time: 24788 ns/iter; 1.7234x vs baseline; 1.7234x over previous
import jax
import jax.numpy as jnp
from jax import lax
from jax.experimental import pallas as pl
from jax.experimental.pallas import tpu as pltpu

N_DEV = 32


def kernel(Q, K, V):
    b, q_len, h, d = Q.shape
    kv_len = K.shape[1]
    bh = b * h
    scale = d ** -0.5
    pack = 128

    Q3 = Q.reshape(b, h, d)
    Kt = K.transpose(0, 2, 3, 1)
    Vt = V.transpose(0, 2, 3, 1)

    def body(q_ref, k_ref, v_ref, o_ref, send_ref, allrecv_ref, send_sems, recv_sems):
        my = lax.axis_index("i")

        barrier_sem = pltpu.get_barrier_semaphore()
        for off in range(1, N_DEV):
            pl.semaphore_signal(
                barrier_sem,
                inc=1,
                device_id=((my + off) % N_DEV,),
                device_id_type=pl.DeviceIdType.MESH,
            )

        f32 = jnp.float32

        qB = q_ref[...] * scale
        S = jnp.sum(k_ref[...] * qB[..., None], axis=2)
        m3 = jnp.max(S, axis=2, keepdims=True)
        p = jnp.exp(S - m3)
        l3 = jnp.sum(p, axis=2, keepdims=True)
        o3 = jnp.sum(v_ref[...] * p[:, :, None, :], axis=3)

        o = o3.reshape(bh, d)
        m = m3.reshape(bh, 1)
        l = l3.reshape(bh, 1)

        send_ref[:, 0:d] = o.astype(jnp.bfloat16)
        send_ref[:, d:d + 1] = m.astype(jnp.bfloat16)
        send_ref[:, d + 1:d + 2] = l.astype(jnp.bfloat16)

        pl.semaphore_wait(barrier_sem, N_DEV - 1)

        sends = []
        for off in range(1, N_DEV):
            rdma = pltpu.make_async_remote_copy(
                src_ref=send_ref,
                dst_ref=allrecv_ref.at[my],
                send_sem=send_sems.at[off],
                recv_sem=recv_sems.at[my],
                device_id=((my + off) % N_DEV,),
                device_id_type=pl.DeviceIdType.MESH,
            )
            rdma.start()
            sends.append(rdma)

        allrecv_ref[my] = send_ref[...]

        for off in range(1, N_DEV):
            src = (my + off) % N_DEV
            recv = pltpu.make_async_remote_copy(
                src_ref=send_ref,
                dst_ref=allrecv_ref.at[src],
                send_sem=send_sems.at[off],
                recv_sem=recv_sems.at[src],
                device_id=(src,),
                device_id_type=pl.DeviceIdType.MESH,
            )
            recv.wait_recv()

        data = allrecv_ref[...]
        om = data[:, :, 0:d].astype(f32)
        mm = data[:, :, d:d + 1].astype(f32)
        lm = data[:, :, d + 1:d + 2].astype(f32)

        mg = jnp.max(mm, axis=0)
        a = jnp.exp(mm - mg[None, :, :])
        lg = jnp.sum(lm * a, axis=0)
        og = jnp.sum(om * a, axis=0)
        o_ref[...] = og / lg

        for rdma in sends:
            rdma.wait_send()

    out2d = pl.pallas_call(
        body,
        out_shape=jax.ShapeDtypeStruct((bh, d), jnp.float32),
        in_specs=[
            pl.BlockSpec(memory_space=pltpu.VMEM),
            pl.BlockSpec(memory_space=pltpu.VMEM),
            pl.BlockSpec(memory_space=pltpu.VMEM),
        ],
        out_specs=pl.BlockSpec(memory_space=pltpu.VMEM),
        scratch_shapes=[
            pltpu.VMEM((bh, pack), jnp.bfloat16),
            pltpu.VMEM((N_DEV, bh, pack), jnp.bfloat16),
            pltpu.SemaphoreType.DMA((N_DEV,)),
            pltpu.SemaphoreType.DMA((N_DEV,)),
        ],
        compiler_params=pltpu.CompilerParams(collective_id=0),
    )(Q3, Kt, Vt)
    return out2d.reshape(b, q_len, h, d)


# device time: 24773 ns/iter; 1.7245x vs baseline; 1.0006x over previous
import jax
import jax.numpy as jnp
from jax import lax
from jax.experimental import pallas as pl
from jax.experimental.pallas import tpu as pltpu

N_DEV = 32


def kernel(Q, K, V):
    b, q_len, h, d = Q.shape
    kv_len = K.shape[1]
    bh = b * h
    scale = d ** -0.5
    pack = 66

    Q3 = Q.reshape(b, h, d)
    Kt = K.transpose(0, 2, 3, 1)
    Vt = V.transpose(0, 2, 3, 1)

    def body(q_ref, k_any, v_any, o_ref,
             kbuf, vbuf, send_ref, allrecv_ref,
             ksem, vsem, send_sems, recv_sems):
        my = lax.axis_index("i")

        barrier_sem = pltpu.get_barrier_semaphore()
        for off in range(1, N_DEV):
            pl.semaphore_signal(
                barrier_sem,
                inc=1,
                device_id=((my + off) % N_DEV,),
                device_id_type=pl.DeviceIdType.MESH,
            )

        kcp = pltpu.make_async_copy(k_any, kbuf, ksem)
        vcp = pltpu.make_async_copy(v_any, vbuf, vsem)
        kcp.start()
        vcp.start()
        kcp.wait()
        vcp.wait()

        f32 = jnp.float32

        qB = q_ref[...] * scale
        S = jnp.sum(kbuf[...] * qB[..., None], axis=2)
        m3 = jnp.max(S, axis=2, keepdims=True)
        p = jnp.exp(S - m3)
        l3 = jnp.sum(p, axis=2, keepdims=True)
        o3 = jnp.sum(vbuf[...] * p[:, :, None, :], axis=3)

        o = o3.reshape(bh, d)
        m = m3.reshape(bh, 1)
        l = l3.reshape(bh, 1)

        send_ref[:, 0:d] = o.astype(jnp.bfloat16)
        send_ref[:, d:d + 1] = m.astype(jnp.bfloat16)
        send_ref[:, d + 1:d + 2] = l.astype(jnp.bfloat16)

        pl.semaphore_wait(barrier_sem, N_DEV - 1)

        sends = []
        for off in range(1, N_DEV):
            rdma = pltpu.make_async_remote_copy(
                src_ref=send_ref,
                dst_ref=allrecv_ref.at[my],
                send_sem=send_sems.at[off],
                recv_sem=recv_sems.at[my],
                device_id=((my + off) % N_DEV,),
                device_id_type=pl.DeviceIdType.MESH,
            )
            rdma.start()
            sends.append(rdma)

        allrecv_ref[my] = send_ref[...]

        for off in range(1, N_DEV):
            src = (my + off) % N_DEV
            recv = pltpu.make_async_remote_copy(
                src_ref=send_ref,
                dst_ref=allrecv_ref.at[src],
                send_sem=send_sems.at[off],
                recv_sem=recv_sems.at[src],
                device_id=(src,),
                device_id_type=pl.DeviceIdType.MESH,
            )
            recv.wait_recv()

        data = allrecv_ref[...]
        om = data[:, :, 0:d].astype(f32)
        mm = data[:, :, d:d + 1].astype(f32)
        lm = data[:, :, d + 1:d + 2].astype(f32)

        mg = jnp.max(mm, axis=0)
        a = jnp.exp(mm - mg[None, :, :])
        lg = jnp.sum(lm * a, axis=0)
        og = jnp.sum(om * a, axis=0)
        o_ref[...] = og / lg

        for rdma in sends:
            rdma.wait_send()

    out2d = pl.pallas_call(
        body,
        out_shape=jax.ShapeDtypeStruct((bh, d), jnp.float32),
        in_specs=[
            pl.BlockSpec(memory_space=pltpu.VMEM),
            pl.BlockSpec(memory_space=pl.ANY),
            pl.BlockSpec(memory_space=pl.ANY),
        ],
        out_specs=pl.BlockSpec(memory_space=pltpu.VMEM),
        scratch_shapes=[
            pltpu.VMEM((b, h, d, kv_len), jnp.float32),
            pltpu.VMEM((b, h, d, kv_len), jnp.float32),
            pltpu.VMEM((bh, pack), jnp.bfloat16),
            pltpu.VMEM((N_DEV, bh, pack), jnp.bfloat16),
            pltpu.SemaphoreType.DMA,
            pltpu.SemaphoreType.DMA,
            pltpu.SemaphoreType.DMA((N_DEV,)),
            pltpu.SemaphoreType.DMA((N_DEV,)),
        ],
        compiler_params=pltpu.CompilerParams(collective_id=0),
    )(Q3, Kt, Vt)
    return out2d.reshape(b, q_len, h, d)
